# baseline (speedup 1.0000x reference)
import jax
import jax.numpy as jnp
from jax import lax
from jax.experimental import pallas as pl
from jax.experimental.pallas import tpu as pltpu
from jax.experimental.pallas import tpu_sc as plsc

B = 256
N = 111
NP = 128
DEG = 16
EPG = N * DEG
D2 = 128
K1 = 56
K2 = 28
NSC = 32
GPW = B // NSC


def _adj_body(e0_hbm, e1_hbm, out_hbm,
              e0_v0, e1_v0, a_v0, e0_v1, e1_v1, a_v1, sem_in, sem_out):
    wid = lax.axis_index("s") * 2 + lax.axis_index("c")
    base = wid * GPW
    bufs = ((e0_v0, e1_v0, a_v0), (e0_v1, e1_v1, a_v1))

    def start_in(gi):
        e0_v, e1_v, _ = bufs[gi % 2]
        g = base + gi
        return (
            pltpu.async_copy(e0_hbm.at[pl.ds(g * EPG, EPG)], e0_v, sem_in),
            pltpu.async_copy(e1_hbm.at[pl.ds(g * EPG, EPG)], e1_v, sem_in),
        )

    in_handles = {0: start_in(0)}
    out_handles = [None, None]
    zeros16 = jnp.zeros((16,), jnp.float32)
    ones16 = jnp.ones((16,), jnp.float32)

    for gi in range(GPW):
        b = gi % 2
        e0_v, e1_v, a_v = bufs[b]
        g = base + gi
        if gi + 1 < GPW:
            in_handles[gi + 1] = start_in(gi + 1)
        for h in in_handles.pop(gi):
            h.wait()
        if out_handles[b] is not None:
            out_handles[b].wait()

        def zbody(i, c):
            for u in range(8):
                a_v[i, pl.ds(u * 16, 16)] = zeros16
            return c
        lax.fori_loop(0, NP, zbody, 0)

        def ebody(i, c):
            for u in range(3):
                k = i * 3 + u
                e0 = e0_v[pl.ds(k * 16, 16)]
                e1 = e1_v[pl.ds(k * 16, 16)]
                s = lax.rem(e0, N)
                d = lax.rem(e1, N)
                plsc.store_scatter(a_v, [s, d], ones16)
                plsc.store_scatter(a_v, [d, s], ones16)
            return c
        lax.fori_loop(0, EPG // (16 * 3), ebody, 0)

        out_handles[b] = pltpu.async_copy(a_v, out_hbm.at[g], sem_out)

    for h in out_handles:
        h.wait()


def _build_adj(e0, e1):
    mesh = plsc.VectorSubcoreMesh(
        core_axis_name="c", subcore_axis_name="s", num_cores=2, num_subcores=16
    )
    f = pl.kernel(
        _adj_body,
        out_type=jax.ShapeDtypeStruct((B, NP, NP), jnp.float32),
        mesh=mesh,
        compiler_params=pltpu.CompilerParams(needs_layout_passes=False),
        scratch_types=[
            pltpu.VMEM((EPG,), jnp.int32),
            pltpu.VMEM((EPG,), jnp.int32),
            pltpu.VMEM((NP, NP), jnp.float32),
            pltpu.VMEM((EPG,), jnp.int32),
            pltpu.VMEM((EPG,), jnp.int32),
            pltpu.VMEM((NP, NP), jnp.float32),
            pltpu.SemaphoreType.DMA,
            pltpu.SemaphoreType.DMA,
        ],
    )
    return f(e0, e1)


GPP = 4
KF = 32


def _graph_body(a_ref, x_ref, w1_ref, wsp1_ref, w2_ref, wsp2_ref,
                xf_ref, x1_ref, x2_ref):
    A = a_ref[...]
    X = x_ref[...]
    I = lax.broadcasted_iota(jnp.int32, (1, NP, NP), 1)
    J = lax.broadcasted_iota(jnp.int32, (1, NP, NP), 2)

    def bmm(a, b):
        return lax.dot_general(a, b, (((2,), (1,)), ((0,), (0,))),
                               preferred_element_type=jnp.float32)

    def bmmT(a, b):
        return lax.dot_general(a, b, (((2,), (2,)), ((0,), (0,))),
                               preferred_element_type=jnp.float32)

    def wmm(a3, w):
        g_, n_, k_ = a3.shape
        r = jnp.dot(a3.reshape(g_ * n_, k_), w,
                    preferred_element_type=jnp.float32)
        return r.reshape(g_, n_, w.shape[1])

    def norm_adj(Ab, n):
        At = Ab + jnp.where((I == J) & (I < n), 1.0, 0.0)
        dr = jnp.sum(At, axis=2, keepdims=True)
        dc = jnp.sum(At, axis=1, keepdims=True)
        return (At * lax.rsqrt(jnp.maximum(dr, 1e-12))
                   * lax.rsqrt(jnp.maximum(dc, 1e-12)))

    def pool(Xc, Anc, Ac, n, k, want_a):
        AX = bmm(Anc, Xc)
        sc = jnp.sum(jnp.abs(Xc - AX), axis=2, keepdims=True)
        ivalid = lax.broadcasted_iota(jnp.int32, (1, NP, 1), 1) < n
        sc = jnp.where(ivalid, sc, -1e30)
        scT = jnp.transpose(sc, (0, 2, 1))
        C = (sc > scT) | ((sc == scT) & (I < J))
        rankT = jnp.sum(C.astype(jnp.float32), axis=1, keepdims=True)
        P = jnp.where((I.astype(jnp.float32) == rankT) & (I < k), 1.0, 0.0)
        Xp = bmm(P, Xc)
        if not want_a:
            return Xp, None
        Ap = bmmT(bmm(P, Ac), P)
        return Xp, Ap

    An = norm_adj(A, N)
    agg = bmm(An, X)
    xm = jnp.maximum(wmm(agg, w1_ref[...]), 0.0)
    xp = jnp.maximum(wmm(agg, wsp1_ref[...]), 0.0)
    X2 = jnp.concatenate([xm, xp], axis=2)

    Xp1, Ap1 = pool(X2, An, A, N, K1, True)
    x1max = jnp.max(Xp1, axis=1, keepdims=True)
    x1mean = jnp.sum(Xp1, axis=1, keepdims=True) / K1

    An1 = norm_adj(Ap1, K1)
    agg1 = bmm(An1, Xp1)
    xm2 = jnp.maximum(wmm(agg1, w2_ref[...]), 0.0)
    xp2 = jnp.maximum(wmm(agg1, wsp2_ref[...]), 0.0)
    X3 = jnp.concatenate([xm2, xp2], axis=2)

    Xp2, _ = pool(X3, An1, Ap1, K1, K2, False)

    xf_ref[...] = Xp2[:, 0:KF, :]
    x1_ref[...] = jnp.concatenate([x1max, x1mean], axis=2)
    x2_ref[...] = jnp.concatenate(
        [jnp.max(Xp2, axis=1, keepdims=True),
         jnp.sum(Xp2, axis=1, keepdims=True) / K2], axis=2)


def _graph_pipeline(Ab, Xp, W1p, Wsp1p, W2, Wsp2):
    return pl.pallas_call(
        _graph_body,
        grid=(B // GPP,),
        in_specs=[
            pl.BlockSpec((GPP, NP, NP), lambda i: (i, 0, 0)),
            pl.BlockSpec((GPP, NP, NP), lambda i: (i, 0, 0)),
            pl.BlockSpec((NP, D2), lambda i: (0, 0)),
            pl.BlockSpec((NP, D2), lambda i: (0, 0)),
            pl.BlockSpec((2 * D2, D2), lambda i: (0, 0)),
            pl.BlockSpec((2 * D2, D2), lambda i: (0, 0)),
        ],
        out_specs=[
            pl.BlockSpec((GPP, KF, 2 * D2), lambda i: (i, 0, 0)),
            pl.BlockSpec((GPP, 1, 4 * D2), lambda i: (i, 0, 0)),
            pl.BlockSpec((GPP, 1, 4 * D2), lambda i: (i, 0, 0)),
        ],
        out_shape=[
            jax.ShapeDtypeStruct((B, KF, 2 * D2), jnp.float32),
            jax.ShapeDtypeStruct((B, 1, 4 * D2), jnp.float32),
            jax.ShapeDtypeStruct((B, 1, 4 * D2), jnp.float32),
        ],
    )(Ab, Xp, W1p, Wsp1p, W2, Wsp2)


def _mlp_body(xf_ref, x1_ref, x2_ref, wa_ref, wb_ref, wc_ref,
              bl1_ref, wl2_ref, bl2_ref, out1_ref, out2_ref):
    h = jnp.dot(xf_ref[:, 0:K2 * 2 * D2], wa_ref[...],
                preferred_element_type=jnp.float32)
    h = h + jnp.dot(x1_ref[...], wb_ref[...], preferred_element_type=jnp.float32)
    h = h + jnp.dot(x2_ref[...], wc_ref[...], preferred_element_type=jnp.float32)
    f1 = jnp.maximum(h + bl1_ref[...], 0.0)
    f2 = jnp.dot(f1, wl2_ref[...], preferred_element_type=jnp.float32)
    f2 = jnp.maximum(f2 + bl2_ref[...], 0.0)
    m = jnp.max(f2, axis=1, keepdims=True)
    e = jnp.exp(f2 - m)
    out1_ref[...] = e / jnp.sum(e, axis=1, keepdims=True)
    out2_ref[...] = f2


def _mlp(Xf, x1, x2, Wa, Wb, Wc, bl1, Wl2, bl2):
    nhid = Wl2.shape[0]
    nout = Wl2.shape[1]
    return pl.pallas_call(
        _mlp_body,
        out_shape=[
            jax.ShapeDtypeStruct((B, nout), jnp.float32),
            jax.ShapeDtypeStruct((B, nout), jnp.float32),
        ],
    )(Xf, x1, x2, Wa, Wb, Wc, bl1, Wl2, bl2)


def kernel(x, edge_index, batch, W1, Wsp1, W2, Wsp2, Wl1, bl1, Wl2, bl2):
    e0 = edge_index[0].astype(jnp.int32)
    e1 = edge_index[1].astype(jnp.int32)
    Ab = _build_adj(e0, e1)
    Xp = jnp.pad(x.reshape(B, N, N), ((0, 0), (0, NP - N), (0, NP - N)))
    W1p = jnp.pad(W1, ((0, NP - N), (0, 0)))
    Wsp1p = jnp.pad(Wsp1, ((0, NP - N), (0, 0)))
    Xf, x1, x2 = _graph_pipeline(Ab, Xp, W1p, Wsp1p, W2, Wsp2)
    DIMF = K2 * 2 * D2
    Wa = Wl1[:DIMF]
    Wb = Wl1[DIMF:DIMF + 4 * D2]
    Wc = Wl1[DIMF + 4 * D2:]
    x1 = x1.reshape(B, 4 * D2)
    x2 = x2.reshape(B, 4 * D2)
    out1, out2 = _mlp(Xf.reshape(B, KF * 2 * D2), x1, x2, Wa, Wb, Wc,
                      bl1.reshape(1, -1), Wl2, bl2.reshape(1, -1))
    return (out1, out2)


# speedup vs baseline: 1.1812x; 1.1812x over previous
import jax
import jax.numpy as jnp
from jax import lax
from jax.experimental import pallas as pl
from jax.experimental.pallas import tpu as pltpu
from jax.experimental.pallas import tpu_sc as plsc

B = 256
N = 111
NP = 128
DEG = 16
EPG = N * DEG
D2 = 128
K1 = 56
K2 = 28
NSC = 32
GPW = B // NSC


def _adj_body(e0_hbm, e1_hbm, out_hbm,
              e0_v0, e1_v0, a_v0, e0_v1, e1_v1, a_v1, sem_in, sem_out):
    wid = lax.axis_index("s") * 2 + lax.axis_index("c")
    base = wid * GPW
    bufs = ((e0_v0, e1_v0, a_v0), (e0_v1, e1_v1, a_v1))

    def start_in(gi):
        e0_v, e1_v, _ = bufs[gi % 2]
        g = base + gi
        return (
            pltpu.async_copy(e0_hbm.at[pl.ds(g * EPG, EPG)], e0_v, sem_in),
            pltpu.async_copy(e1_hbm.at[pl.ds(g * EPG, EPG)], e1_v, sem_in),
        )

    in_handles = {0: start_in(0)}
    out_handles = [None, None]
    zeros16 = jnp.zeros((16,), jnp.float32)
    ones16 = jnp.ones((16,), jnp.float32)

    for gi in range(GPW):
        b = gi % 2
        e0_v, e1_v, a_v = bufs[b]
        g = base + gi
        if gi + 1 < GPW:
            in_handles[gi + 1] = start_in(gi + 1)
        for h in in_handles.pop(gi):
            h.wait()
        if out_handles[b] is not None:
            out_handles[b].wait()

        def zbody(i, c):
            for u in range(8):
                a_v[i, pl.ds(u * 16, 16)] = zeros16
            return c
        lax.fori_loop(0, NP, zbody, 0)

        def ebody(i, c):
            for u in range(3):
                k = i * 3 + u
                e0 = e0_v[pl.ds(k * 16, 16)]
                e1 = e1_v[pl.ds(k * 16, 16)]
                s = lax.rem(e0, N)
                d = lax.rem(e1, N)
                plsc.store_scatter(a_v, [s, d], ones16)
                plsc.store_scatter(a_v, [d, s], ones16)
            return c
        lax.fori_loop(0, EPG // (16 * 3), ebody, 0)

        out_handles[b] = pltpu.async_copy(a_v, out_hbm.at[g], sem_out)

    for h in out_handles:
        h.wait()


def _build_adj(e0, e1):
    mesh = plsc.VectorSubcoreMesh(
        core_axis_name="c", subcore_axis_name="s", num_cores=2, num_subcores=16
    )
    f = pl.kernel(
        _adj_body,
        out_type=jax.ShapeDtypeStruct((B, NP, NP), jnp.float32),
        mesh=mesh,
        compiler_params=pltpu.CompilerParams(needs_layout_passes=False),
        scratch_types=[
            pltpu.VMEM((EPG,), jnp.int32),
            pltpu.VMEM((EPG,), jnp.int32),
            pltpu.VMEM((NP, NP), jnp.float32),
            pltpu.VMEM((EPG,), jnp.int32),
            pltpu.VMEM((EPG,), jnp.int32),
            pltpu.VMEM((NP, NP), jnp.float32),
            pltpu.SemaphoreType.DMA,
            pltpu.SemaphoreType.DMA,
        ],
    )
    return f(e0, e1)


GPP = 8
KF = 32


def _graph_body(a_ref, x_ref, w1_ref, wsp1_ref, w2_ref, wsp2_ref,
                xf_ref, x1_ref, x2_ref):
    A = a_ref[...]
    X = x_ref[...]
    I = lax.broadcasted_iota(jnp.int32, (1, NP, NP), 1)
    J = lax.broadcasted_iota(jnp.int32, (1, NP, NP), 2)

    def bmm(a, b):
        return lax.dot_general(a, b, (((2,), (1,)), ((0,), (0,))),
                               preferred_element_type=jnp.float32)

    def bmmT(a, b):
        return lax.dot_general(a, b, (((2,), (2,)), ((0,), (0,))),
                               preferred_element_type=jnp.float32)

    def wmm(a3, w):
        g_, n_, k_ = a3.shape
        r = jnp.dot(a3.reshape(g_ * n_, k_), w,
                    preferred_element_type=jnp.float32)
        return r.reshape(g_, n_, w.shape[1])

    def norm_adj(Ab, n):
        At = Ab + jnp.where((I == J) & (I < n), 1.0, 0.0)
        dr = jnp.sum(At, axis=2, keepdims=True)
        dc = jnp.sum(At, axis=1, keepdims=True)
        return (At * lax.rsqrt(jnp.maximum(dr, 1e-12))
                   * lax.rsqrt(jnp.maximum(dc, 1e-12)))

    def pool(Xc, Anc, Ac, n, k, want_a):
        AX = bmm(Anc, Xc)
        sc = jnp.sum(jnp.abs(Xc - AX), axis=2, keepdims=True)
        ivalid = lax.broadcasted_iota(jnp.int32, (1, NP, 1), 1) < n
        sc = jnp.where(ivalid, sc, -1e30)
        scT = jnp.transpose(sc, (0, 2, 1))
        C = (sc > scT) | ((sc == scT) & (I < J))
        rankT = jnp.sum(C.astype(jnp.float32), axis=1, keepdims=True)
        P = jnp.where((I.astype(jnp.float32) == rankT) & (I < k), 1.0, 0.0)
        Xp = bmm(P, Xc)
        if not want_a:
            return Xp, None
        Ap = bmmT(bmm(P, Ac), P)
        return Xp, Ap

    An = norm_adj(A, N)
    agg = bmm(An, X)
    xm = jnp.maximum(wmm(agg, w1_ref[...]), 0.0)
    xp = jnp.maximum(wmm(agg, wsp1_ref[...]), 0.0)
    X2 = jnp.concatenate([xm, xp], axis=2)

    Xp1, Ap1 = pool(X2, An, A, N, K1, True)
    x1max = jnp.max(Xp1, axis=1, keepdims=True)
    x1mean = jnp.sum(Xp1, axis=1, keepdims=True) / K1

    An1 = norm_adj(Ap1, K1)
    agg1 = bmm(An1, Xp1)
    xm2 = jnp.maximum(wmm(agg1, w2_ref[...]), 0.0)
    xp2 = jnp.maximum(wmm(agg1, wsp2_ref[...]), 0.0)
    X3 = jnp.concatenate([xm2, xp2], axis=2)

    Xp2, _ = pool(X3, An1, Ap1, K1, K2, False)

    xf_ref[...] = Xp2[:, 0:KF, :]
    x1_ref[...] = jnp.concatenate([x1max, x1mean], axis=2)
    x2_ref[...] = jnp.concatenate(
        [jnp.max(Xp2, axis=1, keepdims=True),
         jnp.sum(Xp2, axis=1, keepdims=True) / K2], axis=2)


def _graph_pipeline(Ab, Xp, W1p, Wsp1p, W2, Wsp2):
    return pl.pallas_call(
        _graph_body,
        grid=(B // GPP,),
        in_specs=[
            pl.BlockSpec((GPP, NP, NP), lambda i: (i, 0, 0)),
            pl.BlockSpec((GPP, NP, NP), lambda i: (i, 0, 0)),
            pl.BlockSpec((NP, D2), lambda i: (0, 0)),
            pl.BlockSpec((NP, D2), lambda i: (0, 0)),
            pl.BlockSpec((2 * D2, D2), lambda i: (0, 0)),
            pl.BlockSpec((2 * D2, D2), lambda i: (0, 0)),
        ],
        out_specs=[
            pl.BlockSpec((GPP, KF, 2 * D2), lambda i: (i, 0, 0)),
            pl.BlockSpec((GPP, 1, 4 * D2), lambda i: (i, 0, 0)),
            pl.BlockSpec((GPP, 1, 4 * D2), lambda i: (i, 0, 0)),
        ],
        out_shape=[
            jax.ShapeDtypeStruct((B, KF, 2 * D2), jnp.float32),
            jax.ShapeDtypeStruct((B, 1, 4 * D2), jnp.float32),
            jax.ShapeDtypeStruct((B, 1, 4 * D2), jnp.float32),
        ],
    )(Ab, Xp, W1p, Wsp1p, W2, Wsp2)


def _mlp_body(xf_ref, x1_ref, x2_ref, wa_ref, wb_ref, wc_ref,
              bl1_ref, wl2_ref, bl2_ref, out1_ref, out2_ref):
    h = jnp.dot(xf_ref[:, 0:K2 * 2 * D2], wa_ref[...],
                preferred_element_type=jnp.float32)
    h = h + jnp.dot(x1_ref[...], wb_ref[...], preferred_element_type=jnp.float32)
    h = h + jnp.dot(x2_ref[...], wc_ref[...], preferred_element_type=jnp.float32)
    f1 = jnp.maximum(h + bl1_ref[...], 0.0)
    f2 = jnp.dot(f1, wl2_ref[...], preferred_element_type=jnp.float32)
    f2 = jnp.maximum(f2 + bl2_ref[...], 0.0)
    m = jnp.max(f2, axis=1, keepdims=True)
    e = jnp.exp(f2 - m)
    out1_ref[...] = e / jnp.sum(e, axis=1, keepdims=True)
    out2_ref[...] = f2


def _mlp(Xf, x1, x2, Wa, Wb, Wc, bl1, Wl2, bl2):
    nhid = Wl2.shape[0]
    nout = Wl2.shape[1]
    return pl.pallas_call(
        _mlp_body,
        out_shape=[
            jax.ShapeDtypeStruct((B, nout), jnp.float32),
            jax.ShapeDtypeStruct((B, nout), jnp.float32),
        ],
    )(Xf, x1, x2, Wa, Wb, Wc, bl1, Wl2, bl2)


def kernel(x, edge_index, batch, W1, Wsp1, W2, Wsp2, Wl1, bl1, Wl2, bl2):
    e0 = edge_index[0].astype(jnp.int32)
    e1 = edge_index[1].astype(jnp.int32)
    Ab = _build_adj(e0, e1)
    Xp = jnp.pad(x.reshape(B, N, N), ((0, 0), (0, NP - N), (0, NP - N)))
    W1p = jnp.pad(W1, ((0, NP - N), (0, 0)))
    Wsp1p = jnp.pad(Wsp1, ((0, NP - N), (0, 0)))
    Xf, x1, x2 = _graph_pipeline(Ab, Xp, W1p, Wsp1p, W2, Wsp2)
    DIMF = K2 * 2 * D2
    Wa = Wl1[:DIMF]
    Wb = Wl1[DIMF:DIMF + 4 * D2]
    Wc = Wl1[DIMF + 4 * D2:]
    x1 = x1.reshape(B, 4 * D2)
    x2 = x2.reshape(B, 4 * D2)
    out1, out2 = _mlp(Xf.reshape(B, KF * 2 * D2), x1, x2, Wa, Wb, Wc,
                      bl1.reshape(1, -1), Wl2, bl2.reshape(1, -1))
    return (out1, out2)


# speedup vs baseline: 1.2586x; 1.0655x over previous
import jax
import jax.numpy as jnp
from jax import lax
from jax.experimental import pallas as pl
from jax.experimental.pallas import tpu as pltpu
from jax.experimental.pallas import tpu_sc as plsc

B = 256
N = 111
NP = 128
DEG = 16
EPG = N * DEG
D2 = 128
K1 = 56
K2 = 28
NSC = 32
GPW = B // NSC


def _adj_body(e0_hbm, e1_hbm, out_hbm,
              e0_v0, e1_v0, a_v0, e0_v1, e1_v1, a_v1, sem_in, sem_out):
    wid = lax.axis_index("s") * 2 + lax.axis_index("c")
    base = wid * GPW
    bufs = ((e0_v0, e1_v0, a_v0), (e0_v1, e1_v1, a_v1))

    def start_in(gi):
        e0_v, e1_v, _ = bufs[gi % 2]
        g = base + gi
        return (
            pltpu.async_copy(e0_hbm.at[pl.ds(g * EPG, EPG)], e0_v, sem_in),
            pltpu.async_copy(e1_hbm.at[pl.ds(g * EPG, EPG)], e1_v, sem_in),
        )

    in_handles = {0: start_in(0)}
    out_handles = [None, None]
    zeros16 = jnp.zeros((16,), jnp.float32)
    ones16 = jnp.ones((16,), jnp.float32)

    for gi in range(GPW):
        b = gi % 2
        e0_v, e1_v, a_v = bufs[b]
        g = base + gi
        if gi + 1 < GPW:
            in_handles[gi + 1] = start_in(gi + 1)
        for h in in_handles.pop(gi):
            h.wait()
        if out_handles[b] is not None:
            out_handles[b].wait()

        def zbody(i, c):
            for u in range(8):
                a_v[i, pl.ds(u * 16, 16)] = zeros16
            return c
        lax.fori_loop(0, NP, zbody, 0)

        def ebody(i, c):
            for u in range(3):
                k = i * 3 + u
                e0 = e0_v[pl.ds(k * 16, 16)]
                e1 = e1_v[pl.ds(k * 16, 16)]
                s = lax.rem(e0, N)
                d = lax.rem(e1, N)
                plsc.store_scatter(a_v, [s, d], ones16)
                plsc.store_scatter(a_v, [d, s], ones16)
            return c
        lax.fori_loop(0, EPG // (16 * 3), ebody, 0)

        out_handles[b] = pltpu.async_copy(a_v, out_hbm.at[g], sem_out)

    for h in out_handles:
        h.wait()


def _build_adj(e0, e1):
    mesh = plsc.VectorSubcoreMesh(
        core_axis_name="c", subcore_axis_name="s", num_cores=2, num_subcores=16
    )
    f = pl.kernel(
        _adj_body,
        out_type=jax.ShapeDtypeStruct((B, NP, NP), jnp.float32),
        mesh=mesh,
        compiler_params=pltpu.CompilerParams(needs_layout_passes=False),
        scratch_types=[
            pltpu.VMEM((EPG,), jnp.int32),
            pltpu.VMEM((EPG,), jnp.int32),
            pltpu.VMEM((NP, NP), jnp.float32),
            pltpu.VMEM((EPG,), jnp.int32),
            pltpu.VMEM((EPG,), jnp.int32),
            pltpu.VMEM((NP, NP), jnp.float32),
            pltpu.SemaphoreType.DMA,
            pltpu.SemaphoreType.DMA,
        ],
    )
    return f(e0, e1)


GPP = 16
KF = 32


def _graph_body(a_ref, x_ref, w1_ref, wsp1_ref, w2_ref, wsp2_ref,
                xf_ref, x1_ref, x2_ref):
    A = a_ref[...]
    X = x_ref[...]
    I = lax.broadcasted_iota(jnp.int32, (1, NP, NP), 1)
    J = lax.broadcasted_iota(jnp.int32, (1, NP, NP), 2)

    def bmm(a, b):
        return lax.dot_general(a, b, (((2,), (1,)), ((0,), (0,))),
                               preferred_element_type=jnp.float32)

    def bmmT(a, b):
        return lax.dot_general(a, b, (((2,), (2,)), ((0,), (0,))),
                               preferred_element_type=jnp.float32)

    def wmm(a3, w):
        g_, n_, k_ = a3.shape
        r = jnp.dot(a3.reshape(g_ * n_, k_), w,
                    preferred_element_type=jnp.float32)
        return r.reshape(g_, n_, w.shape[1])

    def norm_adj(Ab, n):
        At = Ab + jnp.where((I == J) & (I < n), 1.0, 0.0)
        dr = jnp.sum(At, axis=2, keepdims=True)
        dc = jnp.sum(At, axis=1, keepdims=True)
        return (At * lax.rsqrt(jnp.maximum(dr, 1e-12))
                   * lax.rsqrt(jnp.maximum(dc, 1e-12)))

    def pool(Xc, Anc, Ac, n, k, want_a):
        AX = bmm(Anc, Xc)
        sc = jnp.sum(jnp.abs(Xc - AX), axis=2, keepdims=True)
        ivalid = lax.broadcasted_iota(jnp.int32, (1, NP, 1), 1) < n
        sc = jnp.where(ivalid, sc, -1e30)
        scT = jnp.transpose(sc, (0, 2, 1))
        C = (sc > scT) | ((sc == scT) & (I < J))
        rankT = jnp.sum(C.astype(jnp.float32), axis=1, keepdims=True)
        P = jnp.where((I.astype(jnp.float32) == rankT) & (I < k), 1.0, 0.0)
        Xp = bmm(P, Xc)
        if not want_a:
            return Xp, None
        Ap = bmmT(bmm(P, Ac), P)
        return Xp, Ap

    An = norm_adj(A, N)
    agg = bmm(An, X)
    xm = jnp.maximum(wmm(agg, w1_ref[...]), 0.0)
    xp = jnp.maximum(wmm(agg, wsp1_ref[...]), 0.0)
    X2 = jnp.concatenate([xm, xp], axis=2)

    Xp1, Ap1 = pool(X2, An, A, N, K1, True)
    x1max = jnp.max(Xp1, axis=1, keepdims=True)
    x1mean = jnp.sum(Xp1, axis=1, keepdims=True) / K1

    An1 = norm_adj(Ap1, K1)
    agg1 = bmm(An1, Xp1)
    xm2 = jnp.maximum(wmm(agg1, w2_ref[...]), 0.0)
    xp2 = jnp.maximum(wmm(agg1, wsp2_ref[...]), 0.0)
    X3 = jnp.concatenate([xm2, xp2], axis=2)

    Xp2, _ = pool(X3, An1, Ap1, K1, K2, False)

    xf_ref[...] = Xp2[:, 0:KF, :]
    x1_ref[...] = jnp.concatenate([x1max, x1mean], axis=2)
    x2_ref[...] = jnp.concatenate(
        [jnp.max(Xp2, axis=1, keepdims=True),
         jnp.sum(Xp2, axis=1, keepdims=True) / K2], axis=2)


def _graph_pipeline(Ab, Xp, W1p, Wsp1p, W2, Wsp2):
    return pl.pallas_call(
        _graph_body,
        grid=(B // GPP,),
        in_specs=[
            pl.BlockSpec((GPP, NP, NP), lambda i: (i, 0, 0)),
            pl.BlockSpec((GPP, NP, NP), lambda i: (i, 0, 0)),
            pl.BlockSpec((NP, D2), lambda i: (0, 0)),
            pl.BlockSpec((NP, D2), lambda i: (0, 0)),
            pl.BlockSpec((2 * D2, D2), lambda i: (0, 0)),
            pl.BlockSpec((2 * D2, D2), lambda i: (0, 0)),
        ],
        out_specs=[
            pl.BlockSpec((GPP, KF, 2 * D2), lambda i: (i, 0, 0)),
            pl.BlockSpec((GPP, 1, 4 * D2), lambda i: (i, 0, 0)),
            pl.BlockSpec((GPP, 1, 4 * D2), lambda i: (i, 0, 0)),
        ],
        out_shape=[
            jax.ShapeDtypeStruct((B, KF, 2 * D2), jnp.float32),
            jax.ShapeDtypeStruct((B, 1, 4 * D2), jnp.float32),
            jax.ShapeDtypeStruct((B, 1, 4 * D2), jnp.float32),
        ],
    )(Ab, Xp, W1p, Wsp1p, W2, Wsp2)


def _mlp_body(xf_ref, x1_ref, x2_ref, wa_ref, wb_ref, wc_ref,
              bl1_ref, wl2_ref, bl2_ref, out1_ref, out2_ref):
    h = jnp.dot(xf_ref[:, 0:K2 * 2 * D2], wa_ref[...],
                preferred_element_type=jnp.float32)
    h = h + jnp.dot(x1_ref[...], wb_ref[...], preferred_element_type=jnp.float32)
    h = h + jnp.dot(x2_ref[...], wc_ref[...], preferred_element_type=jnp.float32)
    f1 = jnp.maximum(h + bl1_ref[...], 0.0)
    f2 = jnp.dot(f1, wl2_ref[...], preferred_element_type=jnp.float32)
    f2 = jnp.maximum(f2 + bl2_ref[...], 0.0)
    m = jnp.max(f2, axis=1, keepdims=True)
    e = jnp.exp(f2 - m)
    out1_ref[...] = e / jnp.sum(e, axis=1, keepdims=True)
    out2_ref[...] = f2


def _mlp(Xf, x1, x2, Wa, Wb, Wc, bl1, Wl2, bl2):
    nhid = Wl2.shape[0]
    nout = Wl2.shape[1]
    return pl.pallas_call(
        _mlp_body,
        out_shape=[
            jax.ShapeDtypeStruct((B, nout), jnp.float32),
            jax.ShapeDtypeStruct((B, nout), jnp.float32),
        ],
    )(Xf, x1, x2, Wa, Wb, Wc, bl1, Wl2, bl2)


def kernel(x, edge_index, batch, W1, Wsp1, W2, Wsp2, Wl1, bl1, Wl2, bl2):
    e0 = edge_index[0].astype(jnp.int32)
    e1 = edge_index[1].astype(jnp.int32)
    Ab = _build_adj(e0, e1)
    Xp = jnp.pad(x.reshape(B, N, N), ((0, 0), (0, NP - N), (0, NP - N)))
    W1p = jnp.pad(W1, ((0, NP - N), (0, 0)))
    Wsp1p = jnp.pad(Wsp1, ((0, NP - N), (0, 0)))
    Xf, x1, x2 = _graph_pipeline(Ab, Xp, W1p, Wsp1p, W2, Wsp2)
    DIMF = K2 * 2 * D2
    Wa = Wl1[:DIMF]
    Wb = Wl1[DIMF:DIMF + 4 * D2]
    Wc = Wl1[DIMF + 4 * D2:]
    x1 = x1.reshape(B, 4 * D2)
    x2 = x2.reshape(B, 4 * D2)
    out1, out2 = _mlp(Xf.reshape(B, KF * 2 * D2), x1, x2, Wa, Wb, Wc,
                      bl1.reshape(1, -1), Wl2, bl2.reshape(1, -1))
    return (out1, out2)


# speedup vs baseline: 1.2878x; 1.0232x over previous
import jax
import jax.numpy as jnp
from jax import lax
from jax.experimental import pallas as pl
from jax.experimental.pallas import tpu as pltpu
from jax.experimental.pallas import tpu_sc as plsc

B = 256
N = 111
NP = 128
DEG = 16
EPG = N * DEG
D2 = 128
K1 = 56
K2 = 28
NSC = 32
GPW = B // NSC


def _adj_body(e0_hbm, e1_hbm, out_hbm,
              e0_v0, e1_v0, a_v0, e0_v1, e1_v1, a_v1, sem_in, sem_out):
    wid = lax.axis_index("s") * 2 + lax.axis_index("c")
    base = wid * GPW
    bufs = ((e0_v0, e1_v0, a_v0), (e0_v1, e1_v1, a_v1))

    def start_in(gi):
        e0_v, e1_v, _ = bufs[gi % 2]
        g = base + gi
        return (
            pltpu.async_copy(e0_hbm.at[pl.ds(g * EPG, EPG)], e0_v, sem_in),
            pltpu.async_copy(e1_hbm.at[pl.ds(g * EPG, EPG)], e1_v, sem_in),
        )

    in_handles = {0: start_in(0)}
    out_handles = [None, None]
    zeros16 = jnp.zeros((16,), jnp.float32)
    ones16 = jnp.ones((16,), jnp.float32)

    for gi in range(GPW):
        b = gi % 2
        e0_v, e1_v, a_v = bufs[b]
        g = base + gi
        if gi + 1 < GPW:
            in_handles[gi + 1] = start_in(gi + 1)
        for h in in_handles.pop(gi):
            h.wait()
        if out_handles[b] is not None:
            out_handles[b].wait()

        def zbody(i, c):
            for u in range(8):
                a_v[i, pl.ds(u * 16, 16)] = zeros16
            return c
        lax.fori_loop(0, NP, zbody, 0)

        def ebody(i, c):
            for u in range(3):
                k = i * 3 + u
                e0 = e0_v[pl.ds(k * 16, 16)]
                e1 = e1_v[pl.ds(k * 16, 16)]
                s = lax.rem(e0, N)
                d = lax.rem(e1, N)
                plsc.store_scatter(a_v, [s, d], ones16)
                plsc.store_scatter(a_v, [d, s], ones16)
            return c
        lax.fori_loop(0, EPG // (16 * 3), ebody, 0)

        out_handles[b] = pltpu.async_copy(a_v, out_hbm.at[g], sem_out)

    for h in out_handles:
        h.wait()


def _build_adj(e0, e1):
    mesh = plsc.VectorSubcoreMesh(
        core_axis_name="c", subcore_axis_name="s", num_cores=2, num_subcores=16
    )
    f = pl.kernel(
        _adj_body,
        out_type=jax.ShapeDtypeStruct((B, NP, NP), jnp.float32),
        mesh=mesh,
        compiler_params=pltpu.CompilerParams(needs_layout_passes=False),
        scratch_types=[
            pltpu.VMEM((EPG,), jnp.int32),
            pltpu.VMEM((EPG,), jnp.int32),
            pltpu.VMEM((NP, NP), jnp.float32),
            pltpu.VMEM((EPG,), jnp.int32),
            pltpu.VMEM((EPG,), jnp.int32),
            pltpu.VMEM((NP, NP), jnp.float32),
            pltpu.SemaphoreType.DMA,
            pltpu.SemaphoreType.DMA,
        ],
    )
    return f(e0, e1)


GPP = 32
KF = 32


def _graph_body(a_ref, x_ref, w1_ref, wsp1_ref, w2_ref, wsp2_ref,
                xf_ref, x1_ref, x2_ref):
    A = a_ref[...]
    X = x_ref[...]
    I = lax.broadcasted_iota(jnp.int32, (1, NP, NP), 1)
    J = lax.broadcasted_iota(jnp.int32, (1, NP, NP), 2)

    def bmm(a, b):
        return lax.dot_general(a, b, (((2,), (1,)), ((0,), (0,))),
                               preferred_element_type=jnp.float32)

    def bmmT(a, b):
        return lax.dot_general(a, b, (((2,), (2,)), ((0,), (0,))),
                               preferred_element_type=jnp.float32)

    def wmm(a3, w):
        g_, n_, k_ = a3.shape
        r = jnp.dot(a3.reshape(g_ * n_, k_), w,
                    preferred_element_type=jnp.float32)
        return r.reshape(g_, n_, w.shape[1])

    def norm_adj(Ab, n):
        At = Ab + jnp.where((I == J) & (I < n), 1.0, 0.0)
        dr = jnp.sum(At, axis=2, keepdims=True)
        dc = jnp.sum(At, axis=1, keepdims=True)
        return (At * lax.rsqrt(jnp.maximum(dr, 1e-12))
                   * lax.rsqrt(jnp.maximum(dc, 1e-12)))

    def pool(Xc, Anc, Ac, n, k, want_a):
        AX = bmm(Anc, Xc)
        sc = jnp.sum(jnp.abs(Xc - AX), axis=2, keepdims=True)
        ivalid = lax.broadcasted_iota(jnp.int32, (1, NP, 1), 1) < n
        sc = jnp.where(ivalid, sc, -1e30)
        scT = jnp.transpose(sc, (0, 2, 1))
        C = (sc > scT) | ((sc == scT) & (I < J))
        rankT = jnp.sum(C.astype(jnp.float32), axis=1, keepdims=True)
        P = jnp.where((I.astype(jnp.float32) == rankT) & (I < k), 1.0, 0.0)
        Xp = bmm(P, Xc)
        if not want_a:
            return Xp, None
        Ap = bmmT(bmm(P, Ac), P)
        return Xp, Ap

    An = norm_adj(A, N)
    agg = bmm(An, X)
    xm = jnp.maximum(wmm(agg, w1_ref[...]), 0.0)
    xp = jnp.maximum(wmm(agg, wsp1_ref[...]), 0.0)
    X2 = jnp.concatenate([xm, xp], axis=2)

    Xp1, Ap1 = pool(X2, An, A, N, K1, True)
    x1max = jnp.max(Xp1, axis=1, keepdims=True)
    x1mean = jnp.sum(Xp1, axis=1, keepdims=True) / K1

    An1 = norm_adj(Ap1, K1)
    agg1 = bmm(An1, Xp1)
    xm2 = jnp.maximum(wmm(agg1, w2_ref[...]), 0.0)
    xp2 = jnp.maximum(wmm(agg1, wsp2_ref[...]), 0.0)
    X3 = jnp.concatenate([xm2, xp2], axis=2)

    Xp2, _ = pool(X3, An1, Ap1, K1, K2, False)

    xf_ref[...] = Xp2[:, 0:KF, :]
    x1_ref[...] = jnp.concatenate([x1max, x1mean], axis=2)
    x2_ref[...] = jnp.concatenate(
        [jnp.max(Xp2, axis=1, keepdims=True),
         jnp.sum(Xp2, axis=1, keepdims=True) / K2], axis=2)


def _graph_pipeline(Ab, Xp, W1p, Wsp1p, W2, Wsp2):
    return pl.pallas_call(
        _graph_body,
        grid=(B // GPP,),
        in_specs=[
            pl.BlockSpec((GPP, NP, NP), lambda i: (i, 0, 0)),
            pl.BlockSpec((GPP, NP, NP), lambda i: (i, 0, 0)),
            pl.BlockSpec((NP, D2), lambda i: (0, 0)),
            pl.BlockSpec((NP, D2), lambda i: (0, 0)),
            pl.BlockSpec((2 * D2, D2), lambda i: (0, 0)),
            pl.BlockSpec((2 * D2, D2), lambda i: (0, 0)),
        ],
        out_specs=[
            pl.BlockSpec((GPP, KF, 2 * D2), lambda i: (i, 0, 0)),
            pl.BlockSpec((GPP, 1, 4 * D2), lambda i: (i, 0, 0)),
            pl.BlockSpec((GPP, 1, 4 * D2), lambda i: (i, 0, 0)),
        ],
        out_shape=[
            jax.ShapeDtypeStruct((B, KF, 2 * D2), jnp.float32),
            jax.ShapeDtypeStruct((B, 1, 4 * D2), jnp.float32),
            jax.ShapeDtypeStruct((B, 1, 4 * D2), jnp.float32),
        ],
    )(Ab, Xp, W1p, Wsp1p, W2, Wsp2)


def _mlp_body(xf_ref, x1_ref, x2_ref, wa_ref, wb_ref, wc_ref,
              bl1_ref, wl2_ref, bl2_ref, out1_ref, out2_ref):
    h = jnp.dot(xf_ref[:, 0:K2 * 2 * D2], wa_ref[...],
                preferred_element_type=jnp.float32)
    h = h + jnp.dot(x1_ref[...], wb_ref[...], preferred_element_type=jnp.float32)
    h = h + jnp.dot(x2_ref[...], wc_ref[...], preferred_element_type=jnp.float32)
    f1 = jnp.maximum(h + bl1_ref[...], 0.0)
    f2 = jnp.dot(f1, wl2_ref[...], preferred_element_type=jnp.float32)
    f2 = jnp.maximum(f2 + bl2_ref[...], 0.0)
    m = jnp.max(f2, axis=1, keepdims=True)
    e = jnp.exp(f2 - m)
    out1_ref[...] = e / jnp.sum(e, axis=1, keepdims=True)
    out2_ref[...] = f2


def _mlp(Xf, x1, x2, Wa, Wb, Wc, bl1, Wl2, bl2):
    nhid = Wl2.shape[0]
    nout = Wl2.shape[1]
    return pl.pallas_call(
        _mlp_body,
        out_shape=[
            jax.ShapeDtypeStruct((B, nout), jnp.float32),
            jax.ShapeDtypeStruct((B, nout), jnp.float32),
        ],
    )(Xf, x1, x2, Wa, Wb, Wc, bl1, Wl2, bl2)


def kernel(x, edge_index, batch, W1, Wsp1, W2, Wsp2, Wl1, bl1, Wl2, bl2):
    e0 = edge_index[0].astype(jnp.int32)
    e1 = edge_index[1].astype(jnp.int32)
    Ab = _build_adj(e0, e1)
    Xp = jnp.pad(x.reshape(B, N, N), ((0, 0), (0, NP - N), (0, NP - N)))
    W1p = jnp.pad(W1, ((0, NP - N), (0, 0)))
    Wsp1p = jnp.pad(Wsp1, ((0, NP - N), (0, 0)))
    Xf, x1, x2 = _graph_pipeline(Ab, Xp, W1p, Wsp1p, W2, Wsp2)
    DIMF = K2 * 2 * D2
    Wa = Wl1[:DIMF]
    Wb = Wl1[DIMF:DIMF + 4 * D2]
    Wc = Wl1[DIMF + 4 * D2:]
    x1 = x1.reshape(B, 4 * D2)
    x2 = x2.reshape(B, 4 * D2)
    out1, out2 = _mlp(Xf.reshape(B, KF * 2 * D2), x1, x2, Wa, Wb, Wc,
                      bl1.reshape(1, -1), Wl2, bl2.reshape(1, -1))
    return (out1, out2)


# speedup vs baseline: 1.3337x; 1.0356x over previous
import jax
import jax.numpy as jnp
from jax import lax
from jax.experimental import pallas as pl
from jax.experimental.pallas import tpu as pltpu
from jax.experimental.pallas import tpu_sc as plsc

B = 256
N = 111
NP = 128
DEG = 16
EPG = N * DEG
D2 = 128
K1 = 56
K2 = 28
NSC = 32
GPW = B // NSC


def _make_adj_body(gpw):
  def _adj_body(e0_hbm, e1_hbm, out_hbm,
                e0_v0, e1_v0, a_v0, e0_v1, e1_v1, a_v1, sem_in, sem_out):
    wid = lax.axis_index("s") * 2 + lax.axis_index("c")
    base = wid * gpw
    bufs = ((e0_v0, e1_v0, a_v0), (e0_v1, e1_v1, a_v1))

    def start_in(gi):
        e0_v, e1_v, _ = bufs[gi % 2]
        g = base + gi
        return (
            pltpu.async_copy(e0_hbm.at[pl.ds(g * EPG, EPG)], e0_v, sem_in),
            pltpu.async_copy(e1_hbm.at[pl.ds(g * EPG, EPG)], e1_v, sem_in),
        )

    in_handles = {0: start_in(0)}
    out_handles = [None, None]
    zeros16 = jnp.zeros((16,), jnp.float32)
    ones16 = jnp.ones((16,), jnp.float32)

    for gi in range(gpw):
        b = gi % 2
        e0_v, e1_v, a_v = bufs[b]
        g = base + gi
        if gi + 1 < gpw:
            in_handles[gi + 1] = start_in(gi + 1)
        for h in in_handles.pop(gi):
            h.wait()
        if out_handles[b] is not None:
            out_handles[b].wait()

        def zbody(i, c):
            for u in range(8):
                a_v[i, pl.ds(u * 16, 16)] = zeros16
            return c
        lax.fori_loop(0, NP, zbody, 0)

        def ebody(i, c):
            for u in range(3):
                k = i * 3 + u
                e0 = e0_v[pl.ds(k * 16, 16)]
                e1 = e1_v[pl.ds(k * 16, 16)]
                s = lax.rem(e0, N)
                d = lax.rem(e1, N)
                plsc.store_scatter(a_v, [s, d], ones16)
                plsc.store_scatter(a_v, [d, s], ones16)
            return c
        lax.fori_loop(0, EPG // (16 * 3), ebody, 0)

        out_handles[b] = pltpu.async_copy(a_v, out_hbm.at[g], sem_out)

    for h in out_handles:
        if h is not None:
            h.wait()
  return _adj_body


def _build_adj(e0, e1, nb):
    mesh = plsc.VectorSubcoreMesh(
        core_axis_name="c", subcore_axis_name="s", num_cores=2, num_subcores=16
    )
    f = pl.kernel(
        _make_adj_body(nb // NSC),
        out_type=jax.ShapeDtypeStruct((nb, NP, NP), jnp.float32),
        mesh=mesh,
        compiler_params=pltpu.CompilerParams(needs_layout_passes=False),
        scratch_types=[
            pltpu.VMEM((EPG,), jnp.int32),
            pltpu.VMEM((EPG,), jnp.int32),
            pltpu.VMEM((NP, NP), jnp.float32),
            pltpu.VMEM((EPG,), jnp.int32),
            pltpu.VMEM((EPG,), jnp.int32),
            pltpu.VMEM((NP, NP), jnp.float32),
            pltpu.SemaphoreType.DMA,
            pltpu.SemaphoreType.DMA,
        ],
    )
    return f(e0, e1)


GPP = 32
KF = 32


def _graph_body(a_ref, x_ref, w1_ref, wsp1_ref, w2_ref, wsp2_ref,
                xf_ref, x1_ref, x2_ref):
    A = a_ref[...]
    X = x_ref[...]
    I = lax.broadcasted_iota(jnp.int32, (1, NP, NP), 1)
    J = lax.broadcasted_iota(jnp.int32, (1, NP, NP), 2)

    def bmm(a, b):
        return lax.dot_general(a, b, (((2,), (1,)), ((0,), (0,))),
                               preferred_element_type=jnp.float32)

    def bmmT(a, b):
        return lax.dot_general(a, b, (((2,), (2,)), ((0,), (0,))),
                               preferred_element_type=jnp.float32)

    def wmm(a3, w):
        g_, n_, k_ = a3.shape
        r = jnp.dot(a3.reshape(g_ * n_, k_), w,
                    preferred_element_type=jnp.float32)
        return r.reshape(g_, n_, w.shape[1])

    def norm_adj(Ab, n):
        At = Ab + jnp.where((I == J) & (I < n), 1.0, 0.0)
        dr = jnp.sum(At, axis=2, keepdims=True)
        dc = jnp.sum(At, axis=1, keepdims=True)
        return (At * lax.rsqrt(jnp.maximum(dr, 1e-12))
                   * lax.rsqrt(jnp.maximum(dc, 1e-12)))

    def pool(Xc, Anc, Ac, n, k, want_a):
        AX = bmm(Anc, Xc)
        sc = jnp.sum(jnp.abs(Xc - AX), axis=2, keepdims=True)
        ivalid = lax.broadcasted_iota(jnp.int32, (1, NP, 1), 1) < n
        sc = jnp.where(ivalid, sc, -1e30)
        scT = jnp.transpose(sc, (0, 2, 1))
        C = (sc > scT) | ((sc == scT) & (I < J))
        rankT = jnp.sum(C.astype(jnp.float32), axis=1, keepdims=True)
        P = jnp.where((I.astype(jnp.float32) == rankT) & (I < k), 1.0, 0.0)
        Xp = bmm(P, Xc)
        if not want_a:
            return Xp, None
        Ap = bmmT(bmm(P, Ac), P)
        return Xp, Ap

    An = norm_adj(A, N)
    agg = bmm(An, X)
    xm = jnp.maximum(wmm(agg, w1_ref[...]), 0.0)
    xp = jnp.maximum(wmm(agg, wsp1_ref[...]), 0.0)
    X2 = jnp.concatenate([xm, xp], axis=2)

    Xp1, Ap1 = pool(X2, An, A, N, K1, True)
    x1max = jnp.max(Xp1, axis=1, keepdims=True)
    x1mean = jnp.sum(Xp1, axis=1, keepdims=True) / K1

    An1 = norm_adj(Ap1, K1)
    agg1 = bmm(An1, Xp1)
    xm2 = jnp.maximum(wmm(agg1, w2_ref[...]), 0.0)
    xp2 = jnp.maximum(wmm(agg1, wsp2_ref[...]), 0.0)
    X3 = jnp.concatenate([xm2, xp2], axis=2)

    Xp2, _ = pool(X3, An1, Ap1, K1, K2, False)

    xf_ref[...] = Xp2[:, 0:KF, :]
    x1_ref[...] = jnp.concatenate([x1max, x1mean], axis=2)
    x2_ref[...] = jnp.concatenate(
        [jnp.max(Xp2, axis=1, keepdims=True),
         jnp.sum(Xp2, axis=1, keepdims=True) / K2], axis=2)


def _graph_pipeline(Ab, Xp, W1p, Wsp1p, W2, Wsp2):
    nb = Ab.shape[0]
    return pl.pallas_call(
        _graph_body,
        grid=(nb // GPP,),
        in_specs=[
            pl.BlockSpec((GPP, NP, NP), lambda i: (i, 0, 0)),
            pl.BlockSpec((GPP, NP, NP), lambda i: (i, 0, 0)),
            pl.BlockSpec((NP, D2), lambda i: (0, 0)),
            pl.BlockSpec((NP, D2), lambda i: (0, 0)),
            pl.BlockSpec((2 * D2, D2), lambda i: (0, 0)),
            pl.BlockSpec((2 * D2, D2), lambda i: (0, 0)),
        ],
        out_specs=[
            pl.BlockSpec((GPP, KF, 2 * D2), lambda i: (i, 0, 0)),
            pl.BlockSpec((GPP, 1, 4 * D2), lambda i: (i, 0, 0)),
            pl.BlockSpec((GPP, 1, 4 * D2), lambda i: (i, 0, 0)),
        ],
        out_shape=[
            jax.ShapeDtypeStruct((nb, KF, 2 * D2), jnp.float32),
            jax.ShapeDtypeStruct((nb, 1, 4 * D2), jnp.float32),
            jax.ShapeDtypeStruct((nb, 1, 4 * D2), jnp.float32),
        ],
    )(Ab, Xp, W1p, Wsp1p, W2, Wsp2)


def _mlp_body(xfa_ref, xfb_ref, x1a_ref, x1b_ref, x2a_ref, x2b_ref,
              wa_ref, wb_ref, wc_ref,
              bl1_ref, wl2_ref, bl2_ref, out1_ref, out2_ref):
    xf = jnp.concatenate([xfa_ref[...], xfb_ref[...]], axis=0)
    x1 = jnp.concatenate([x1a_ref[...], x1b_ref[...]], axis=0)
    x2 = jnp.concatenate([x2a_ref[...], x2b_ref[...]], axis=0)
    h = jnp.dot(xf[:, 0:K2 * 2 * D2], wa_ref[...],
                preferred_element_type=jnp.float32)
    h = h + jnp.dot(x1, wb_ref[...], preferred_element_type=jnp.float32)
    h = h + jnp.dot(x2, wc_ref[...], preferred_element_type=jnp.float32)
    f1 = jnp.maximum(h + bl1_ref[...], 0.0)
    f2 = jnp.dot(f1, wl2_ref[...], preferred_element_type=jnp.float32)
    f2 = jnp.maximum(f2 + bl2_ref[...], 0.0)
    m = jnp.max(f2, axis=1, keepdims=True)
    e = jnp.exp(f2 - m)
    out1_ref[...] = e / jnp.sum(e, axis=1, keepdims=True)
    out2_ref[...] = f2


def _mlp(Xfa, Xfb, x1a, x1b, x2a, x2b, Wa, Wb, Wc, bl1, Wl2, bl2):
    nhid = Wl2.shape[0]
    nout = Wl2.shape[1]
    return pl.pallas_call(
        _mlp_body,
        out_shape=[
            jax.ShapeDtypeStruct((B, nout), jnp.float32),
            jax.ShapeDtypeStruct((B, nout), jnp.float32),
        ],
    )(Xfa, Xfb, x1a, x1b, x2a, x2b, Wa, Wb, Wc, bl1, Wl2, bl2)


def kernel(x, edge_index, batch, W1, Wsp1, W2, Wsp2, Wl1, bl1, Wl2, bl2):
    e0 = edge_index[0].astype(jnp.int32)
    e1 = edge_index[1].astype(jnp.int32)
    half = B // 2
    eh = half * EPG
    x3 = x.reshape(B, N, N)
    W1p = jnp.pad(W1, ((0, NP - N), (0, 0)))
    Wsp1p = jnp.pad(Wsp1, ((0, NP - N), (0, 0)))
    Ab1 = _build_adj(e0[:eh], e1[:eh], half)
    Ab2 = _build_adj(e0[eh:], e1[eh:], half)
    pad = ((0, 0), (0, NP - N), (0, NP - N))
    Xp1 = jnp.pad(x3[:half], pad)
    Xp2 = jnp.pad(x3[half:], pad)
    Xfa, x1a, x2a = _graph_pipeline(Ab1, Xp1, W1p, Wsp1p, W2, Wsp2)
    Xfb, x1b, x2b = _graph_pipeline(Ab2, Xp2, W1p, Wsp1p, W2, Wsp2)
    DIMF = K2 * 2 * D2
    Wa = Wl1[:DIMF]
    Wb = Wl1[DIMF:DIMF + 4 * D2]
    Wc = Wl1[DIMF + 4 * D2:]
    out1, out2 = _mlp(Xfa.reshape(half, KF * 2 * D2),
                      Xfb.reshape(half, KF * 2 * D2),
                      x1a.reshape(half, 4 * D2), x1b.reshape(half, 4 * D2),
                      x2a.reshape(half, 4 * D2), x2b.reshape(half, 4 * D2),
                      Wa, Wb, Wc, bl1.reshape(1, -1), Wl2, bl2.reshape(1, -1))
    return (out1, out2)
